# baseline (speedup 1.0000x reference)
import jax
import jax.numpy as jnp
from jax import lax
from jax.experimental import pallas as pl
from jax.experimental.pallas import tpu as pltpu

_N = 128
_HEADS = 4
_HID = 32
_OUT_DIM = 2
_NB = 4


def _gat_clique(h, asd_m, mask4, ones_bd, maskd, outd):
    asd = jnp.dot(h, asd_m)
    als = lax.slice(asd, (0, 0), (_N, _HEADS))
    ald = lax.slice(asd, (0, _HEADS), (_N, 2 * _HEADS))
    als_t = als.T
    als_tile = jnp.concatenate([als_t] * _HEADS, axis=1)
    rhs = jnp.concatenate([mask4, als_tile * mask4], axis=0)
    lhs = jnp.concatenate([ald, jnp.ones_like(ald)], axis=1)
    e = jnp.dot(lhs, rhs)
    e = jnp.maximum(e, 0.2 * e)
    alsmax = jnp.max(als, axis=0, keepdims=True)
    mt = ald + alsmax
    mcol = jnp.maximum(mt, 0.2 * mt)
    mb = jnp.dot(mcol, mask4)
    ex = jnp.exp(e - mb)
    den = jnp.dot(ex, ones_bd)
    r = jnp.dot(1.0 / (den + 1e-16), maskd)
    oh = []
    for hd in range(_HEADS):
        ex_h = lax.slice(ex, (0, hd * _N), (_N, (hd + 1) * _N))
        h_h = lax.slice(h, (0, hd * outd), (_N, (hd + 1) * outd))
        oh.append(jnp.dot(ex_h, h_h))
    return jnp.concatenate(oh, axis=1) * r


def _block_kernel(xs_ref, pe_ref, mask_ref, onesbd_ref, maskh_ref, masko_ref,
                  w1_ref, asd1_ref, b1_ref,
                  w2_ref, asd2_ref, b2_ref,
                  w3_ref, asd3_ref, b3_ref, out_ref):
    mask4 = mask_ref[...]
    ones_bd = onesbd_ref[...]
    maskh = maskh_ref[...]
    masko = masko_ref[...]
    w1 = w1_ref[...]
    pew = jnp.dot(pe_ref[0], w1)
    w1r0 = lax.slice(w1, (0, 0), (1, _N))
    xst = xs_ref[0, 0, 0].T
    o1 = []
    for b in range(_NB):
        xcol = lax.slice(xst, (0, b), (_N, b + 1))
        h1 = pew + jnp.dot(xcol, w1r0)
        o1.append(_gat_clique(h1, asd1_ref[...], mask4, ones_bd, maskh, _HID))
    o1 = jnp.concatenate(o1, axis=0) + b1_ref[...]
    h2 = jnp.dot(o1, w2_ref[...])
    o2 = []
    for b in range(_NB):
        h2b = lax.slice(h2, (b * _N, 0), ((b + 1) * _N, _HEADS * _HID))
        o2.append(_gat_clique(h2b, asd2_ref[...], mask4, ones_bd, maskh, _HID))
    o2 = jnp.concatenate(o2, axis=0) + b2_ref[...]
    h3 = jnp.dot(o2, w3_ref[...])
    for b in range(_NB):
        h3b = lax.slice(h3, (b * _N, 0), ((b + 1) * _N, _HEADS * _OUT_DIM))
        o3b = _gat_clique(h3b, asd3_ref[...], mask4, ones_bd, masko, _OUT_DIM)
        o3b = o3b + b3_ref[...]
        out_ref[b, 0, :] = jnp.mean(o3b, axis=0)


def _attn_mat(a_src, a_dst):
    heads, outd = a_src.shape
    eye = jnp.eye(heads, dtype=a_src.dtype)
    s = (eye[:, :, None] * a_src[None, :, :]).reshape(heads, heads * outd).T
    d = (eye[:, :, None] * a_dst[None, :, :]).reshape(heads, heads * outd).T
    return jnp.concatenate([s, d], axis=1)


def kernel(xs, pos_enc, W1, a_src1, a_dst1, b1, W2, a_src2, a_dst2, b2,
           W3, a_src3, a_dst3, b3):
    bs, nr, nc = xs.shape
    enc = pos_enc.shape[-1]
    nblocks = bs * nr
    steps_per_graph = nr // _NB
    pe_pad = jnp.pad(pos_enc, ((0, 0), (0, 0), (1, 0)))
    xs4 = xs.reshape(bs, steps_per_graph, 1, _NB, nc)
    eye4 = jnp.eye(_HEADS, dtype=jnp.float32)
    mask4 = jnp.repeat(eye4, _N, axis=1)
    ones_bd = jnp.repeat(eye4, _N, axis=0)
    maskh = jnp.repeat(eye4, _HID, axis=1)
    masko = jnp.repeat(eye4, _OUT_DIM, axis=1)

    def whole(shape):
        return pl.BlockSpec(shape, lambda i: tuple(0 for _ in shape))

    hh = _HEADS * _HID
    ho = _HEADS * _OUT_DIM
    out = pl.pallas_call(
        _block_kernel,
        grid=(nblocks // _NB,),
        in_specs=[
            pl.BlockSpec((1, 1, 1, _NB, nc),
                         lambda i: (i // steps_per_graph, i % steps_per_graph,
                                    0, 0, 0)),
            pl.BlockSpec((1, nc, 1 + enc),
                         lambda i: (i // steps_per_graph, 0, 0)),
            whole((_HEADS, _HEADS * _N)),
            whole((_HEADS * _N, _HEADS)),
            whole((_HEADS, hh)),
            whole((_HEADS, ho)),
            whole((1 + enc, hh)), whole((hh, 2 * _HEADS)), whole((1, hh)),
            whole((hh, hh)), whole((hh, 2 * _HEADS)), whole((1, hh)),
            whole((hh, ho)), whole((ho, 2 * _HEADS)), whole((1, ho)),
        ],
        out_specs=pl.BlockSpec((_NB, 1, ho), lambda i: (i, 0, 0)),
        out_shape=jax.ShapeDtypeStruct((nblocks, 1, ho), jnp.float32),
        compiler_params=pltpu.CompilerParams(
            dimension_semantics=("parallel",)),
    )(
        xs4, pe_pad, mask4, ones_bd, maskh, masko,
        W1, _attn_mat(a_src1, a_dst1), b1.reshape(1, -1),
        W2, _attn_mat(a_src2, a_dst2), b2.reshape(1, -1),
        W3, _attn_mat(a_src3, a_dst3), b3.reshape(1, -1),
    )
    return out.reshape(bs, nr, _HEADS * _OUT_DIM)


# speedup vs baseline: 1.3174x; 1.3174x over previous
import jax
import jax.numpy as jnp
from jax import lax
from jax.experimental import pallas as pl
from jax.experimental.pallas import tpu as pltpu

_N = 128
_HEADS = 4
_HID = 32
_OUT_DIM = 2
_NB = 4


def _gat_clique(h, asd, mask4, outd):
    als = lax.slice(asd, (0, 0), (_N, _HEADS))
    ald = lax.slice(asd, (0, _HEADS), (_N, 2 * _HEADS))
    ald_t = ald.T
    ald_tile = jnp.concatenate([ald_t] * _HEADS, axis=1)
    rhs = jnp.concatenate([mask4, ald_tile * mask4], axis=0)
    lhs = jnp.concatenate([als, jnp.ones_like(als)], axis=1)
    e_wide = jnp.dot(lhs, rhs)
    alsmax = jnp.max(als, axis=0, keepdims=True).T
    mt = ald_t + alsmax
    m_t = jnp.maximum(mt, 0.2 * mt)
    oh = []
    for hd in range(_HEADS):
        e = lax.slice(e_wide, (0, hd * _N), (_N, (hd + 1) * _N))
        e = jnp.maximum(e, 0.2 * e)
        ex = jnp.exp(e - lax.slice(m_t, (hd, 0), (hd + 1, _N)))
        den = jnp.sum(ex, axis=0, keepdims=True)
        alpha = ex * (1.0 / (den + 1e-16))
        h_h = lax.slice(h, (0, hd * outd), (_N, (hd + 1) * outd))
        oh.append(lax.dot_general(alpha, h_h, (((0,), (0,)), ((), ()))))
    return jnp.concatenate(oh, axis=1)


def _layer(hf, asd_m, mask4, bias, outd):
    asd_f = jnp.dot(hf, asd_m)
    outs = []
    for b in range(_NB):
        r0 = b * _N
        asd = lax.slice(asd_f, (r0, 0), (r0 + _N, 2 * _HEADS))
        hb = lax.slice(hf, (r0, 0), (r0 + _N, _HEADS * outd))
        outs.append(_gat_clique(hb, asd, mask4, outd))
    return jnp.concatenate(outs, axis=0) + bias


def _block_kernel(x_ref, mask_ref, w1_ref, asd1_ref, b1_ref,
                  w2_ref, asd2_ref, b2_ref,
                  w3_ref, asd3_ref, b3_ref, out_ref):
    mask4 = mask_ref[...]
    x = x_ref[...].reshape(_NB * _N, _N)
    h1 = jnp.dot(x, w1_ref[...])
    o1 = _layer(h1, asd1_ref[...], mask4, b1_ref[...], _HID)
    h2 = jnp.dot(o1, w2_ref[...])
    o2 = _layer(h2, asd2_ref[...], mask4, b2_ref[...], _HID)
    h3 = jnp.dot(o2, w3_ref[...])
    o3 = _layer(h3, asd3_ref[...], mask4, b3_ref[...], _OUT_DIM)
    for b in range(_NB):
        blk = lax.slice(o3, (b * _N, 0), ((b + 1) * _N, _HEADS * _OUT_DIM))
        out_ref[b, 0, :] = jnp.mean(blk, axis=0)


def _attn_mat(a_src, a_dst):
    heads, outd = a_src.shape
    eye = jnp.eye(heads, dtype=a_src.dtype)
    s = (eye[:, :, None] * a_src[None, :, :]).reshape(heads, heads * outd).T
    d = (eye[:, :, None] * a_dst[None, :, :]).reshape(heads, heads * outd).T
    return jnp.concatenate([s, d], axis=1)


def kernel(xs, pos_enc, W1, a_src1, a_dst1, b1, W2, a_src2, a_dst2, b2,
           W3, a_src3, a_dst3, b3):
    bs, nr, nc = xs.shape
    enc = pos_enc.shape[-1]
    nblocks = bs * nr
    pe = jnp.broadcast_to(pos_enc[:, None, :, :], (bs, nr, nc, enc))
    x = jnp.concatenate([xs[..., None], pe], axis=-1).reshape(nblocks, nc, 1 + enc)
    mask4 = jnp.repeat(jnp.eye(_HEADS, dtype=jnp.float32), _N, axis=1)

    def whole(shape):
        return pl.BlockSpec(shape, lambda i: tuple(0 for _ in shape))

    hh = _HEADS * _HID
    ho = _HEADS * _OUT_DIM
    out = pl.pallas_call(
        _block_kernel,
        grid=(nblocks // _NB,),
        in_specs=[
            pl.BlockSpec((_NB, nc, 1 + enc), lambda i: (i, 0, 0)),
            whole((_HEADS, _HEADS * _N)),
            whole((1 + enc, hh)), whole((hh, 2 * _HEADS)), whole((1, hh)),
            whole((hh, hh)), whole((hh, 2 * _HEADS)), whole((1, hh)),
            whole((hh, ho)), whole((ho, 2 * _HEADS)), whole((1, ho)),
        ],
        out_specs=pl.BlockSpec((_NB, 1, ho), lambda i: (i, 0, 0)),
        out_shape=jax.ShapeDtypeStruct((nblocks, 1, ho), jnp.float32),
        compiler_params=pltpu.CompilerParams(
            dimension_semantics=("parallel",)),
    )(
        x, mask4,
        W1, _attn_mat(a_src1, a_dst1), b1.reshape(1, -1),
        W2, _attn_mat(a_src2, a_dst2), b2.reshape(1, -1),
        W3, _attn_mat(a_src3, a_dst3), b3.reshape(1, -1),
    )
    return out.reshape(bs, nr, _HEADS * _OUT_DIM)


# speedup vs baseline: 1.6670x; 1.2654x over previous
import jax
import jax.numpy as jnp
from jax import lax
from jax.experimental import pallas as pl
from jax.experimental.pallas import tpu as pltpu

_N = 128
_HEADS = 4
_HID = 32
_OUT_DIM = 2
_NB = 4


def _gat_clique(h, asd, mask4, ones_col, outd):
    als = lax.slice(asd, (0, 0), (_N, _HEADS))
    ald = lax.slice(asd, (0, _HEADS), (_N, 2 * _HEADS))
    als_t = als.T
    als_tile = jnp.concatenate([als_t] * _HEADS, axis=1)
    rhs = jnp.concatenate([mask4, als_tile * mask4], axis=0)
    lhs = jnp.concatenate([ald, jnp.ones_like(ald)], axis=1)
    e = jnp.dot(lhs, rhs)
    e = jnp.maximum(e, 0.2 * e)
    mm = (jnp.max(als, axis=0, keepdims=True) +
          jnp.max(ald, axis=0, keepdims=True))
    m = jnp.maximum(mm, 0.2 * mm)
    oh = []
    for hd in range(_HEADS):
        e_h = lax.slice(e, (0, hd * _N), (_N, (hd + 1) * _N))
        ex = jnp.exp(e_h - lax.slice(m, (0, hd), (1, hd + 1)))
        h_aug = jnp.concatenate(
            [lax.slice(h, (0, hd * outd), (_N, (hd + 1) * outd)), ones_col],
            axis=1)
        o_aug = jnp.dot(ex, h_aug)
        den = lax.slice(o_aug, (0, outd), (_N, outd + 1))
        o_h = lax.slice(o_aug, (0, 0), (_N, outd))
        oh.append(o_h * (1.0 / (den + 1e-16)))
    return jnp.concatenate(oh, axis=1)


def _layer(hf, asd_m, mask4, ones_col, bias, outd):
    asd_f = jnp.dot(hf, asd_m)
    outs = []
    for b in range(_NB):
        r0 = b * _N
        asd = lax.slice(asd_f, (r0, 0), (r0 + _N, 2 * _HEADS))
        hb = lax.slice(hf, (r0, 0), (r0 + _N, _HEADS * outd))
        outs.append(_gat_clique(hb, asd, mask4, ones_col, outd))
    return jnp.concatenate(outs, axis=0) + bias


def _block_kernel(x_ref, mask_ref, w1_ref, asd1_ref, b1_ref,
                  w2_ref, asd2_ref, b2_ref,
                  w3_ref, asd3_ref, b3_ref, out_ref):
    mask4 = mask_ref[...]
    ones_col = jnp.ones((_N, 1), jnp.float32)
    x = x_ref[...].reshape(_NB * _N, _N)
    h1 = jnp.dot(x, w1_ref[...])
    o1 = _layer(h1, asd1_ref[...], mask4, ones_col, b1_ref[...], _HID)
    h2 = jnp.dot(o1, w2_ref[...])
    o2 = _layer(h2, asd2_ref[...], mask4, ones_col, b2_ref[...], _HID)
    h3 = jnp.dot(o2, w3_ref[...])
    o3 = _layer(h3, asd3_ref[...], mask4, ones_col, b3_ref[...], _OUT_DIM)
    for b in range(_NB):
        blk = lax.slice(o3, (b * _N, 0), ((b + 1) * _N, _HEADS * _OUT_DIM))
        out_ref[b, 0, :] = jnp.mean(blk, axis=0)


def _attn_mat(a_src, a_dst):
    heads, outd = a_src.shape
    eye = jnp.eye(heads, dtype=a_src.dtype)
    s = (eye[:, :, None] * a_src[None, :, :]).reshape(heads, heads * outd).T
    d = (eye[:, :, None] * a_dst[None, :, :]).reshape(heads, heads * outd).T
    return jnp.concatenate([s, d], axis=1)


def kernel(xs, pos_enc, W1, a_src1, a_dst1, b1, W2, a_src2, a_dst2, b2,
           W3, a_src3, a_dst3, b3):
    bs, nr, nc = xs.shape
    enc = pos_enc.shape[-1]
    nblocks = bs * nr
    pe = jnp.broadcast_to(pos_enc[:, None, :, :], (bs, nr, nc, enc))
    x = jnp.concatenate([xs[..., None], pe], axis=-1).reshape(nblocks, nc, 1 + enc)
    mask4 = jnp.repeat(jnp.eye(_HEADS, dtype=jnp.float32), _N, axis=1)

    def whole(shape):
        return pl.BlockSpec(shape, lambda i: tuple(0 for _ in shape))

    hh = _HEADS * _HID
    ho = _HEADS * _OUT_DIM
    out = pl.pallas_call(
        _block_kernel,
        grid=(nblocks // _NB,),
        in_specs=[
            pl.BlockSpec((_NB, nc, 1 + enc), lambda i: (i, 0, 0)),
            whole((_HEADS, _HEADS * _N)),
            whole((1 + enc, hh)), whole((hh, 2 * _HEADS)), whole((1, hh)),
            whole((hh, hh)), whole((hh, 2 * _HEADS)), whole((1, hh)),
            whole((hh, ho)), whole((ho, 2 * _HEADS)), whole((1, ho)),
        ],
        out_specs=pl.BlockSpec((_NB, 1, ho), lambda i: (i, 0, 0)),
        out_shape=jax.ShapeDtypeStruct((nblocks, 1, ho), jnp.float32),
        compiler_params=pltpu.CompilerParams(
            dimension_semantics=("parallel",)),
    )(
        x, mask4,
        W1, _attn_mat(a_src1, a_dst1), b1.reshape(1, -1),
        W2, _attn_mat(a_src2, a_dst2), b2.reshape(1, -1),
        W3, _attn_mat(a_src3, a_dst3), b3.reshape(1, -1),
    )
    return out.reshape(bs, nr, _HEADS * _OUT_DIM)


# speedup vs baseline: 1.7782x; 1.0667x over previous
import jax
import jax.numpy as jnp
from jax import lax
from jax.experimental import pallas as pl
from jax.experimental.pallas import tpu as pltpu

_N = 128
_HEADS = 4
_HID = 32
_OUT_DIM = 2
_NB = 8


def _gat_clique(h, asd, mask4, maskf, ones_col, outd):
    als = lax.slice(asd, (0, 0), (_N, _HEADS))
    ald = lax.slice(asd, (0, _HEADS), (_N, 2 * _HEADS))
    als_t = als.T
    als_tile = jnp.concatenate([als_t] * _HEADS, axis=1)
    rhs = jnp.concatenate([mask4, als_tile * mask4], axis=0)
    lhs = jnp.concatenate([ald, jnp.ones_like(ald)], axis=1)
    e = jnp.dot(lhs, rhs)
    e = jnp.maximum(e, 0.2 * e)
    mm = (jnp.max(als, axis=0, keepdims=True) +
          jnp.max(ald, axis=0, keepdims=True))
    m = jnp.maximum(mm, 0.2 * mm)
    oh = []
    for hd in range(_HEADS):
        e_h = lax.slice(e, (0, hd * _N), (_N, (hd + 1) * _N))
        ex = jnp.exp(e_h - lax.slice(m, (0, hd), (1, hd + 1)))
        h_aug = jnp.concatenate(
            [lax.slice(h, (0, hd * outd), (_N, (hd + 1) * outd)), ones_col],
            axis=1)
        o_aug = jnp.dot(ex, h_aug)
        den = lax.slice(o_aug, (0, outd), (_N, outd + 1))
        o_h = lax.slice(o_aug, (0, 0), (_N, outd))
        oh.append(o_h * (1.0 / (den + 1e-16)))
    return jnp.concatenate(oh, axis=1)


def _layer(hf, asd_m, mask4, maskf, ones_col, bias, outd):
    asd_f = jnp.dot(hf, asd_m)
    outs = []
    for b in range(_NB):
        r0 = b * _N
        asd = lax.slice(asd_f, (r0, 0), (r0 + _N, 2 * _HEADS))
        hb = lax.slice(hf, (r0, 0), (r0 + _N, _HEADS * outd))
        outs.append(_gat_clique(hb, asd, mask4, maskf, ones_col, outd))
    return jnp.concatenate(outs, axis=0) + bias


def _block_kernel(xs_ref, pe_ref, mask_ref, maskh_ref, masko_ref,
                  w1_ref, asd1_ref, b1_ref,
                  w2_ref, asd2_ref, b2_ref,
                  w3_ref, asd3_ref, b3_ref, out_ref):
    mask4 = mask_ref[...]
    maskh = maskh_ref[...]
    masko = masko_ref[...]
    ones_col = jnp.ones((_N, 1), jnp.float32)
    w1 = w1_ref[...]
    pew = jnp.dot(pe_ref[0], w1)
    w1r0 = lax.slice(w1, (0, 0), (1, _N))
    xst = xs_ref[0, 0, 0].T
    h1 = jnp.concatenate(
        [pew + jnp.dot(lax.slice(xst, (0, b), (_N, b + 1)), w1r0)
         for b in range(_NB)], axis=0)
    o1 = _layer(h1, asd1_ref[...], mask4, maskh, ones_col, b1_ref[...], _HID)
    h2 = jnp.dot(o1, w2_ref[...])
    o2 = _layer(h2, asd2_ref[...], mask4, maskh, ones_col, b2_ref[...], _HID)
    h3 = jnp.dot(o2, w3_ref[...])
    o3 = _layer(h3, asd3_ref[...], mask4, masko, ones_col, b3_ref[...],
                _OUT_DIM)
    for b in range(_NB):
        blk = lax.slice(o3, (b * _N, 0), ((b + 1) * _N, _HEADS * _OUT_DIM))
        out_ref[b, 0, :] = jnp.mean(blk, axis=0)


def _attn_mat(a_src, a_dst):
    heads, outd = a_src.shape
    eye = jnp.eye(heads, dtype=a_src.dtype)
    s = (eye[:, :, None] * a_src[None, :, :]).reshape(heads, heads * outd).T
    d = (eye[:, :, None] * a_dst[None, :, :]).reshape(heads, heads * outd).T
    return jnp.concatenate([s, d], axis=1)


def kernel(xs, pos_enc, W1, a_src1, a_dst1, b1, W2, a_src2, a_dst2, b2,
           W3, a_src3, a_dst3, b3):
    bs, nr, nc = xs.shape
    enc = pos_enc.shape[-1]
    nblocks = bs * nr
    steps_per_graph = nr // _NB
    pe_pad = jnp.pad(pos_enc, ((0, 0), (0, 0), (1, 0)))
    xs4 = xs.reshape(bs, steps_per_graph, 1, _NB, nc)
    eye4 = jnp.eye(_HEADS, dtype=jnp.float32)
    mask4 = jnp.repeat(eye4, _N, axis=1)
    maskh = jnp.repeat(eye4, _HID, axis=1)
    masko = jnp.repeat(eye4, _OUT_DIM, axis=1)

    def whole(shape):
        return pl.BlockSpec(shape, lambda i: tuple(0 for _ in shape))

    hh = _HEADS * _HID
    ho = _HEADS * _OUT_DIM
    out = pl.pallas_call(
        _block_kernel,
        grid=(nblocks // _NB,),
        in_specs=[
            pl.BlockSpec((1, 1, 1, _NB, nc),
                         lambda i: (i // steps_per_graph, i % steps_per_graph,
                                    0, 0, 0)),
            pl.BlockSpec((1, nc, 1 + enc),
                         lambda i: (i // steps_per_graph, 0, 0)),
            whole((_HEADS, _HEADS * _N)),
            whole((_HEADS, hh)), whole((_HEADS, ho)),
            whole((1 + enc, hh)), whole((hh, 2 * _HEADS)), whole((1, hh)),
            whole((hh, hh)), whole((hh, 2 * _HEADS)), whole((1, hh)),
            whole((hh, ho)), whole((ho, 2 * _HEADS)), whole((1, ho)),
        ],
        out_specs=pl.BlockSpec((_NB, 1, ho), lambda i: (i, 0, 0)),
        out_shape=jax.ShapeDtypeStruct((nblocks, 1, ho), jnp.float32),
        compiler_params=pltpu.CompilerParams(
            dimension_semantics=("parallel",)),
    )(
        xs4, pe_pad, mask4, maskh, masko,
        W1, _attn_mat(a_src1, a_dst1), b1.reshape(1, -1),
        W2, _attn_mat(a_src2, a_dst2), b2.reshape(1, -1),
        W3, _attn_mat(a_src3, a_dst3), b3.reshape(1, -1),
    )
    return out.reshape(bs, nr, _HEADS * _OUT_DIM)
